# baseline (speedup 1.0000x reference)
import functools

import jax
import jax.numpy as jnp
from jax import lax
from jax.experimental import pallas as pl
from jax.experimental.pallas import tpu as pltpu
from jax.experimental.pallas import tpu_sc as plsc

F_CAT = 26
F_NUM = 13
D = 64
EPS = 1e-5


def _make_sc_gather(n_rows: int):
    info = plsc.get_sparse_core_info()
    nc, ns = info.num_cores, info.num_subcores
    nw = nc * ns
    assert n_rows % nw == 0
    n_per_w = n_rows // nw
    CH = 104
    NBUF = 8
    assert n_per_w % CH == 0
    nch = n_per_w // CH
    assert nch % NBUF == 0

    mesh = plsc.VectorSubcoreMesh(core_axis_name="c", subcore_axis_name="s")

    @functools.partial(
        pl.kernel,
        out_type=jax.ShapeDtypeStruct((n_rows, D), jnp.float32),
        mesh=mesh,
        scratch_types=[
            pltpu.VMEM((NBUF, CH), jnp.int32),
            pltpu.VMEM((NBUF, CH, D), jnp.float32),
        ]
        + [pltpu.SemaphoreType.DMA] * (2 * NBUF),
        compiler_params=pltpu.CompilerParams(use_tc_tiling_on_sc=False),
    )
    def sc_gather(table_hbm, idx_hbm, out_hbm, idx_v, rows_v, *sems):
        gsem = sems[:NBUF]
        osem = sems[NBUF:]
        wid = lax.axis_index("s") * nc + lax.axis_index("c")
        base = wid * n_per_w

        def start_gather(b, c):
            off = base + c * CH
            pltpu.sync_copy(idx_hbm.at[pl.ds(off, CH)], idx_v.at[b])
            pltpu.async_copy(table_hbm.at[idx_v.at[b]], rows_v.at[b], gsem[b])

        def wait_gather(b):
            pltpu.make_async_copy(
                table_hbm.at[idx_v.at[b]], rows_v.at[b], gsem[b]
            ).wait()

        def start_out(b, c):
            off = base + c * CH
            pltpu.async_copy(rows_v.at[b], out_hbm.at[pl.ds(off, CH)], osem[b])

        def wait_out(b):
            pltpu.make_async_copy(
                rows_v.at[b], out_hbm.at[pl.ds(base, CH)], osem[b]
            ).wait()

        for b in range(NBUF):
            start_gather(b, b)

        @pl.loop(0, nch - NBUF, step=NBUF)
        def _(g):
            for b in range(NBUF):
                c = g + b
                wait_gather(b)
                start_out(b, c)
                wait_out(b)
                start_gather(b, c + NBUF)

        for b in range(NBUF):
            wait_gather(b)
            start_out(b, nch - NBUF + b)
        for b in range(NBUF):
            wait_out(b)

    return sc_gather


def _epilogue_body(cat_ref, xn_ref, w_ref, b_ref, g_ref, bt_ref, out_ref):
    cat = cat_ref[...]
    xn = xn_ref[...]
    num = xn[:, :, None] * w_ref[...][None] + b_ref[...][None]
    x = jnp.concatenate([cat, num], axis=1)
    mu = jnp.mean(x, axis=-1, keepdims=True)
    xc = x - mu
    var = jnp.mean(xc * xc, axis=-1, keepdims=True)
    y = xc * lax.rsqrt(var + EPS)
    out_ref[...] = y * g_ref[...][None] + bt_ref[...][None]


def _epilogue(cat, x_num, W_num, b_num, gamma, beta):
    B = cat.shape[0]
    BT = 256
    grid = (B // BT,)
    g2 = gamma.reshape(1, D)
    bt2 = beta.reshape(1, D)
    return pl.pallas_call(
        _epilogue_body,
        grid=grid,
        in_specs=[
            pl.BlockSpec((BT, F_CAT, D), lambda i: (i, 0, 0)),
            pl.BlockSpec((BT, F_NUM), lambda i: (i, 0)),
            pl.BlockSpec((F_NUM, D), lambda i: (0, 0)),
            pl.BlockSpec((F_NUM, D), lambda i: (0, 0)),
            pl.BlockSpec((1, D), lambda i: (0, 0)),
            pl.BlockSpec((1, D), lambda i: (0, 0)),
        ],
        out_specs=pl.BlockSpec((BT, F_CAT + F_NUM, D), lambda i: (i, 0, 0)),
        out_shape=jax.ShapeDtypeStruct((B, F_CAT + F_NUM, D), jnp.float32),
        compiler_params=pltpu.CompilerParams(
            dimension_semantics=("parallel",)
        ),
    )(cat, x_num, W_num, b_num, g2, bt2)


def kernel(x_cat, x_num, tables, W_num, b_num, gamma, beta):
    B = x_cat.shape[0]
    rows = tables.shape[1]
    table_flat = tables.reshape(F_CAT * rows, D)
    offs = (jnp.arange(F_CAT, dtype=jnp.int32) * rows)[None, :]
    idx_flat = (x_cat.astype(jnp.int32) + offs).reshape(B * F_CAT)
    cat_flat = _make_sc_gather(B * F_CAT)(table_flat, idx_flat)
    cat = cat_flat.reshape(B, F_CAT, D)
    return _epilogue(cat, x_num, W_num, b_num, gamma, beta)
